# baseline (speedup 1.0000x reference)
import functools

import jax
import jax.numpy as jnp
from jax import lax
from jax.experimental import pallas as pl
from jax.experimental.pallas import tpu as pltpu
from jax.experimental.pallas import tpu_sc as plsc

_NC = 2
_NS = 16
_NW = _NC * _NS
_CHUNK = 128


def _make_sc_gather_sum(n_rows, d, n_pad, rpw):
    n_chunks = rpw // _CHUNK
    mesh = plsc.VectorSubcoreMesh(
        core_axis_name="c", subcore_axis_name="s",
        num_cores=_NC, num_subcores=_NS)

    @functools.partial(
        pl.kernel,
        out_type=jax.ShapeDtypeStruct((n_pad, d), jnp.float32),
        mesh=mesh,
        scratch_types=[
            pltpu.VMEM((_CHUNK,), jnp.int32),
            pltpu.VMEM((_CHUNK,), jnp.int32),
            pltpu.VMEM((_CHUNK,), jnp.int32),
            pltpu.VMEM((_CHUNK, d), jnp.float32),
            pltpu.VMEM((_CHUNK, d), jnp.float32),
            pltpu.VMEM((_CHUNK, d), jnp.float32),
            pltpu.SemaphoreType.DMA,
            pltpu.SemaphoreType.DMA,
            pltpu.SemaphoreType.DMA,
        ],
    )
    def sc_gather_sum(out2_hbm, nb0_hbm, nb1_hbm, nb2_hbm, gsum_hbm,
                      idx0, idx1, idx2, b0, b1, b2, s0, s1, s2):
        wid = lax.axis_index("s") * _NC + lax.axis_index("c")
        base = wid * rpw

        def chunk_body(ci, carry):
            off = base + ci * _CHUNK
            pltpu.sync_copy(nb0_hbm.at[pl.ds(off, _CHUNK)], idx0)
            pltpu.sync_copy(nb1_hbm.at[pl.ds(off, _CHUNK)], idx1)
            pltpu.sync_copy(nb2_hbm.at[pl.ds(off, _CHUNK)], idx2)
            c0 = pltpu.async_copy(out2_hbm.at[idx0], b0, s0)
            c1 = pltpu.async_copy(out2_hbm.at[idx1], b1, s1)
            c2 = pltpu.async_copy(out2_hbm.at[idx2], b2, s2)
            c0.wait()
            c1.wait()
            c2.wait()

            def row_body(r, rc):
                for c in range(d // 16):
                    sl = pl.ds(c * 16, 16)
                    b0[r, sl] = b0[r, sl] + b1[r, sl] + b2[r, sl]
                return rc

            lax.fori_loop(0, _CHUNK, row_body, 0)
            pltpu.sync_copy(b0, gsum_hbm.at[pl.ds(off, _CHUNK)])
            return carry

        lax.fori_loop(0, n_chunks, chunk_body, 0)

    return sc_gather_sum


def _tc_body(o1, o2, g, wc1, wc2, wa, bc, ba, out3, out4):
    a = o1[...].astype(jnp.bfloat16)
    b2f = o2[...]
    b2 = b2f.astype(jnp.bfloat16)
    out3[...] = (
        jnp.dot(a, wc1[...], preferred_element_type=jnp.float32)
        + jnp.dot(b2, wc2[...], preferred_element_type=jnp.float32)
        + bc[...]
    )
    f = ((b2f + g[...]) * 0.25).astype(jnp.bfloat16)
    out4[...] = jnp.dot(f, wa[...], preferred_element_type=jnp.float32) + ba[...]


def kernel(out1, out2, neighbour, W_comb, b_comb, W_agg, b_agg):
    n, d = out2.shape
    dout = b_comb.shape[0]

    rpw = ((n + _NW - 1) // _NW + _CHUNK - 1) // _CHUNK * _CHUNK
    n_pad = _NW * rpw
    nbt = jnp.transpose(neighbour.astype(jnp.int32))
    nbt = jnp.pad(nbt, ((0, 0), (0, n_pad - n)))
    gsum = _make_sc_gather_sum(n, d, n_pad, rpw)(
        out2, nbt[0], nbt[1], nbt[2])[:n]

    wcT = jnp.transpose(W_comb[:, :, 0]).astype(jnp.bfloat16)
    wc1 = wcT[:d]
    wc2 = wcT[d:]
    wa = jnp.transpose(W_agg[:, :, 0]).astype(jnp.bfloat16)
    bc = b_comb.reshape(1, dout)
    ba = b_agg.reshape(1, dout)

    blk = 2000
    assert n % blk == 0
    grid = (n // blk,)
    row_spec = pl.BlockSpec((blk, d), lambda i: (i, 0))
    full = lambda s: pl.BlockSpec(s, lambda i: (0, 0))
    out3, out4 = pl.pallas_call(
        _tc_body,
        grid=grid,
        in_specs=[
            row_spec, row_spec, row_spec,
            full((d, dout)), full((d, dout)), full((d, dout)),
            full((1, dout)), full((1, dout)),
        ],
        out_specs=[
            pl.BlockSpec((blk, dout), lambda i: (i, 0)),
            pl.BlockSpec((blk, dout), lambda i: (i, 0)),
        ],
        out_shape=[
            jax.ShapeDtypeStruct((n, dout), jnp.float32),
            jax.ShapeDtypeStruct((n, dout), jnp.float32),
        ],
    )(out1, out2, gsum, wc1, wc2, wa, bc, ba)
    return (out3, out4)


# speedup vs baseline: 1.1785x; 1.1785x over previous
import functools

import jax
import jax.numpy as jnp
from jax import lax
from jax.experimental import pallas as pl
from jax.experimental.pallas import tpu as pltpu
from jax.experimental.pallas import tpu_sc as plsc

_NC = 2
_NS = 16
_NW = _NC * _NS
_CHUNK = 64


def _make_sc_gather_sum(n_rows, d, n_pad, rpw):
    n_chunks = rpw // _CHUNK
    assert n_chunks % 2 == 0
    mesh = plsc.VectorSubcoreMesh(
        core_axis_name="c", subcore_axis_name="s",
        num_cores=_NC, num_subcores=_NS)

    @functools.partial(
        pl.kernel,
        out_type=jax.ShapeDtypeStruct((n_pad, d), jnp.float32),
        mesh=mesh,
        scratch_types=[
            [pltpu.VMEM((rpw,), jnp.int32) for _ in range(3)],
            [pltpu.VMEM((_CHUNK, d), jnp.float32) for _ in range(3)],
            [pltpu.VMEM((_CHUNK, d), jnp.float32) for _ in range(3)],
            [pltpu.SemaphoreType.DMA for _ in range(2)],
        ],
    )
    def sc_gather_sum(out2_hbm, nb0_hbm, nb1_hbm, nb2_hbm, gsum_hbm,
                      idx_all, bufs0, bufs1, sems):
        bufs = (bufs0, bufs1)
        wid = lax.axis_index("s") * _NC + lax.axis_index("c")
        base = wid * rpw
        pltpu.sync_copy(nb0_hbm.at[pl.ds(base, rpw)], idx_all[0])
        pltpu.sync_copy(nb1_hbm.at[pl.ds(base, rpw)], idx_all[1])
        pltpu.sync_copy(nb2_hbm.at[pl.ds(base, rpw)], idx_all[2])

        def fire(ci, slot):
            for k in range(3):
                idx = idx_all[k].at[pl.ds(ci * _CHUNK, _CHUNK)]
                pltpu.async_copy(out2_hbm.at[idx], bufs[slot][k], sems[slot])

        def drain(slot):
            for k in range(3):
                pltpu.make_async_copy(
                    out2_hbm.at[pl.ds(0, _CHUNK)], bufs[slot][k],
                    sems[slot]).wait()

        def process(ci, slot):
            b0, b1, b2 = bufs[slot]

            def row_body(r, rc):
                for c in range(d // 16):
                    sl = pl.ds(c * 16, 16)
                    b0[r, sl] = b0[r, sl] + b1[r, sl] + b2[r, sl]
                return rc

            lax.fori_loop(0, _CHUNK, row_body, 0)
            pltpu.sync_copy(b0, gsum_hbm.at[pl.ds(base + ci * _CHUNK, _CHUNK)])

        fire(0, 0)

        def pair_body(p, carry):
            ci = 2 * p
            fire(ci + 1, 1)
            drain(0)
            process(ci, 0)

            @pl.when(ci + 2 < n_chunks)
            def _():
                fire(ci + 2, 0)

            drain(1)
            process(ci + 1, 1)
            return carry

        lax.fori_loop(0, n_chunks // 2, pair_body, 0)

    return sc_gather_sum


def _tc_body(o1, o2, g, wc1, wc2, wa, bc, ba, out3, out4):
    a = o1[...].astype(jnp.bfloat16)
    b2f = o2[...]
    b2 = b2f.astype(jnp.bfloat16)
    out3[...] = (
        jnp.dot(a, wc1[...], preferred_element_type=jnp.float32)
        + jnp.dot(b2, wc2[...], preferred_element_type=jnp.float32)
        + bc[...]
    )
    f = ((b2f + g[...]) * 0.25).astype(jnp.bfloat16)
    out4[...] = jnp.dot(f, wa[...], preferred_element_type=jnp.float32) + ba[...]


def kernel(out1, out2, neighbour, W_comb, b_comb, W_agg, b_agg):
    n, d = out2.shape
    dout = b_comb.shape[0]

    step = 2 * _CHUNK
    rpw = ((n + _NW - 1) // _NW + step - 1) // step * step
    n_pad = _NW * rpw
    nbt = jnp.transpose(neighbour.astype(jnp.int32))
    nbt = jnp.pad(nbt, ((0, 0), (0, n_pad - n)))
    gsum = _make_sc_gather_sum(n, d, n_pad, rpw)(
        out2, nbt[0], nbt[1], nbt[2])[:n]

    wcT = jnp.transpose(W_comb[:, :, 0]).astype(jnp.bfloat16)
    wc1 = wcT[:d]
    wc2 = wcT[d:]
    wa = jnp.transpose(W_agg[:, :, 0]).astype(jnp.bfloat16)
    bc = b_comb.reshape(1, dout)
    ba = b_agg.reshape(1, dout)

    blk = 2000
    assert n % blk == 0
    grid = (n // blk,)
    row_spec = pl.BlockSpec((blk, d), lambda i: (i, 0))
    full = lambda s: pl.BlockSpec(s, lambda i: (0, 0))
    out3, out4 = pl.pallas_call(
        _tc_body,
        grid=grid,
        in_specs=[
            row_spec, row_spec, row_spec,
            full((d, dout)), full((d, dout)), full((d, dout)),
            full((1, dout)), full((1, dout)),
        ],
        out_specs=[
            pl.BlockSpec((blk, dout), lambda i: (i, 0)),
            pl.BlockSpec((blk, dout), lambda i: (i, 0)),
        ],
        out_shape=[
            jax.ShapeDtypeStruct((n, dout), jnp.float32),
            jax.ShapeDtypeStruct((n, dout), jnp.float32),
        ],
    )(out1, out2, gsum, wc1, wc2, wa, bc, ba)
    return (out3, out4)


# speedup vs baseline: 1.2737x; 1.0808x over previous
import functools

import jax
import jax.numpy as jnp
from jax import lax
from jax.experimental import pallas as pl
from jax.experimental.pallas import tpu as pltpu
from jax.experimental.pallas import tpu_sc as plsc

_NC = 2
_NS = 16
_NW = _NC * _NS
_CHUNK = 64


def _make_sc_gather_sum(n_rows, d, n_pad, rpw):
    n_chunks = rpw // _CHUNK
    assert n_chunks % 2 == 0
    mesh = plsc.VectorSubcoreMesh(
        core_axis_name="c", subcore_axis_name="s",
        num_cores=_NC, num_subcores=_NS)

    @functools.partial(
        pl.kernel,
        out_type=jax.ShapeDtypeStruct((n_pad, d), jnp.float32),
        mesh=mesh,
        scratch_types=[
            [pltpu.VMEM((rpw,), jnp.int32) for _ in range(3)],
            [pltpu.VMEM((_CHUNK, d), jnp.float32) for _ in range(3)],
            [pltpu.VMEM((_CHUNK, d), jnp.float32) for _ in range(3)],
            [pltpu.SemaphoreType.DMA for _ in range(2)],
        ],
    )
    def sc_gather_sum(out2_hbm, nb0_hbm, nb1_hbm, nb2_hbm, gsum_hbm,
                      idx_all, bufs0, bufs1, sems):
        bufs = (bufs0, bufs1)
        wid = lax.axis_index("c") * _NS + lax.axis_index("s")
        base = wid * rpw
        pltpu.sync_copy(nb0_hbm.at[pl.ds(base, rpw)], idx_all[0])
        pltpu.sync_copy(nb1_hbm.at[pl.ds(base, rpw)], idx_all[1])
        pltpu.sync_copy(nb2_hbm.at[pl.ds(base, rpw)], idx_all[2])

        def fire(ci, slot):
            for k in range(3):
                idx = idx_all[k].at[pl.ds(ci * _CHUNK, _CHUNK)]
                pltpu.async_copy(out2_hbm.at[idx], bufs[slot][k], sems[slot])

        def drain(slot):
            for k in range(3):
                pltpu.make_async_copy(
                    out2_hbm.at[pl.ds(0, _CHUNK)], bufs[slot][k],
                    sems[slot]).wait()

        def process(ci, slot):
            b0, b1, b2 = bufs[slot]

            def row_body(r, rc):
                for c in range(d // 16):
                    sl = pl.ds(c * 16, 16)
                    b0[r, sl] = b0[r, sl] + b1[r, sl] + b2[r, sl]
                return rc

            lax.fori_loop(0, _CHUNK, row_body, 0)
            pltpu.sync_copy(b0, gsum_hbm.at[pl.ds(base + ci * _CHUNK, _CHUNK)])

        fire(0, 0)

        def pair_body(p, carry):
            ci = 2 * p
            fire(ci + 1, 1)
            drain(0)
            process(ci, 0)

            @pl.when(ci + 2 < n_chunks)
            def _():
                fire(ci + 2, 0)

            drain(1)
            process(ci + 1, 1)
            return carry

        lax.fori_loop(0, n_chunks // 2, pair_body, 0)

    return sc_gather_sum


def _tc_body(o1, o2, g, wc1, wc2, wa, bc, ba, out3, out4):
    a = o1[...].astype(jnp.bfloat16)
    b2f = o2[...]
    b2 = b2f.astype(jnp.bfloat16)
    out3[...] = (
        jnp.dot(a, wc1[...], preferred_element_type=jnp.float32)
        + jnp.dot(b2, wc2[...], preferred_element_type=jnp.float32)
        + bc[...]
    )
    f = ((b2f + g[...]) * 0.25).astype(jnp.bfloat16)
    out4[...] = jnp.dot(f, wa[...], preferred_element_type=jnp.float32) + ba[...]


def kernel(out1, out2, neighbour, W_comb, b_comb, W_agg, b_agg):
    n, d = out2.shape
    dout = b_comb.shape[0]

    step = 2 * _CHUNK
    rpw = ((n + _NW - 1) // _NW + step - 1) // step * step
    n_pad = _NW * rpw
    nbt = jnp.transpose(neighbour.astype(jnp.int32))
    nbt = jnp.pad(nbt, ((0, 0), (0, n_pad - n)))
    gsum = _make_sc_gather_sum(n, d, n_pad, rpw)(
        out2, nbt[0], nbt[1], nbt[2])

    wcT = jnp.transpose(W_comb[:, :, 0]).astype(jnp.bfloat16)
    wc1 = wcT[:d]
    wc2 = wcT[d:]
    wa = jnp.transpose(W_agg[:, :, 0]).astype(jnp.bfloat16)
    bc = b_comb.reshape(1, dout)
    ba = b_agg.reshape(1, dout)

    blk = 2000
    assert n % blk == 0
    grid = (n // blk,)
    row_spec = pl.BlockSpec((blk, d), lambda i: (i, 0))
    full = lambda s: pl.BlockSpec(s, lambda i: (0, 0))
    out3, out4 = pl.pallas_call(
        _tc_body,
        grid=grid,
        in_specs=[
            row_spec, row_spec, row_spec,
            full((d, dout)), full((d, dout)), full((d, dout)),
            full((1, dout)), full((1, dout)),
        ],
        out_specs=[
            pl.BlockSpec((blk, dout), lambda i: (i, 0)),
            pl.BlockSpec((blk, dout), lambda i: (i, 0)),
        ],
        out_shape=[
            jax.ShapeDtypeStruct((n, dout), jnp.float32),
            jax.ShapeDtypeStruct((n, dout), jnp.float32),
        ],
    )(out1, out2, gsum, wc1, wc2, wa, bc, ba)
    return (out3, out4)


# speedup vs baseline: 2.0859x; 1.6377x over previous
import functools

import jax
import jax.numpy as jnp
from jax import lax
from jax.experimental import pallas as pl
from jax.experimental.pallas import tpu as pltpu
from jax.experimental.pallas import tpu_sc as plsc

_NC = 2
_NS = 16
_NW = _NC * _NS
_CHUNK = 64


def _make_sc_gather_sum(n_rows, d, n_pad, rpw0, rpw1):
    rpw_max = max(rpw0, rpw1)
    mesh = plsc.VectorSubcoreMesh(
        core_axis_name="c", subcore_axis_name="s",
        num_cores=_NC, num_subcores=_NS)

    @functools.partial(
        pl.kernel,
        out_type=jax.ShapeDtypeStruct((n_pad, d), jnp.float32),
        mesh=mesh,
        scratch_types=[
            [pltpu.VMEM((rpw_max,), jnp.int32) for _ in range(3)],
            [pltpu.VMEM((_CHUNK, d), jnp.float32) for _ in range(3)],
            [pltpu.VMEM((_CHUNK, d), jnp.float32) for _ in range(3)],
            [pltpu.SemaphoreType.DMA for _ in range(2)],
        ],
    )
    def sc_gather_sum(out2_hbm, nb0_hbm, nb1_hbm, nb2_hbm, gsum_hbm,
                      idx_all, bufs0, bufs1, sems):
        bufs = (bufs0, bufs1)
        cid = lax.axis_index("c")
        sid = lax.axis_index("s")
        rpw = lax.select(cid == 0, rpw0, rpw1)
        base = lax.select(cid == 0, sid * rpw0, _NS * rpw0 + sid * rpw1)
        n_chunks = rpw // _CHUNK
        nbs = (nb0_hbm, nb1_hbm, nb2_hbm)
        for k in range(3):
            pltpu.sync_copy(nbs[k].at[pl.ds(base, rpw_max)], idx_all[k])

        def fire(ci, slot):
            for k in range(3):
                idx = idx_all[k].at[pl.ds(ci * _CHUNK, _CHUNK)]
                pltpu.async_copy(out2_hbm.at[idx], bufs[slot][k], sems[slot])

        def drain(slot):
            for k in range(3):
                pltpu.make_async_copy(
                    out2_hbm.at[pl.ds(0, _CHUNK)], bufs[slot][k],
                    sems[slot]).wait()

        def process(ci, slot):
            b0, b1, b2 = bufs[slot]

            def row_body(r, rc):
                for c in range(d // 16):
                    sl = pl.ds(c * 16, 16)
                    b0[r, sl] = b0[r, sl] + b1[r, sl] + b2[r, sl]
                return rc

            lax.fori_loop(0, _CHUNK, row_body, 0)
            pltpu.sync_copy(b0, gsum_hbm.at[pl.ds(base + ci * _CHUNK, _CHUNK)])

        fire(0, 0)

        def pair_body(p, carry):
            ci = 2 * p
            fire(ci + 1, 1)
            drain(0)
            process(ci, 0)

            @pl.when(ci + 2 < n_chunks)
            def _():
                fire(ci + 2, 0)

            drain(1)
            process(ci + 1, 1)
            return carry

        lax.fori_loop(0, n_chunks // 2, pair_body, 0)

    return sc_gather_sum


def _tc3_body(o1, o2, wc1, wc2, bc, out3):
    out3[...] = (
        jnp.dot(o1[...].astype(jnp.bfloat16), wc1[...],
                preferred_element_type=jnp.float32)
        + jnp.dot(o2[...].astype(jnp.bfloat16), wc2[...],
                  preferred_element_type=jnp.float32)
        + bc[...]
    )


def _tc4_body(o2, g, wa, ba, out4):
    f = ((o2[...] + g[...]) * 0.25).astype(jnp.bfloat16)
    out4[...] = jnp.dot(f, wa[...], preferred_element_type=jnp.float32) + ba[...]


def kernel(out1, out2, neighbour, W_comb, b_comb, W_agg, b_agg):
    n, d = out2.shape
    dout = b_comb.shape[0]

    step = 2 * _CHUNK
    per_pair = ((n + _NS - 1) // _NS + step - 1) // step * step
    rpw1 = max(step, int(round(per_pair * 0.265 / step)) * step)
    rpw0 = per_pair - rpw1
    n_pad = _NS * per_pair
    nbt = jnp.transpose(neighbour.astype(jnp.int32))
    nbt = jnp.pad(nbt, ((0, 0), (0, n_pad + max(rpw0, rpw1) - n)))
    gsum = _make_sc_gather_sum(n, d, n_pad, rpw0, rpw1)(
        out2, nbt[0], nbt[1], nbt[2])

    wcT = jnp.transpose(W_comb[:, :, 0]).astype(jnp.bfloat16)
    wc1 = wcT[:d]
    wc2 = wcT[d:]
    wa = jnp.transpose(W_agg[:, :, 0]).astype(jnp.bfloat16)
    bc = b_comb.reshape(1, dout)
    ba = b_agg.reshape(1, dout)

    blk = 2000
    assert n % blk == 0
    grid = (n // blk,)
    row_spec = pl.BlockSpec((blk, d), lambda i: (i, 0))
    out_spec = pl.BlockSpec((blk, dout), lambda i: (i, 0))
    full = lambda s: pl.BlockSpec(s, lambda i: (0, 0))
    out_ty = jax.ShapeDtypeStruct((n, dout), jnp.float32)
    out3 = pl.pallas_call(
        _tc3_body,
        grid=grid,
        in_specs=[row_spec, row_spec, full((d, dout)), full((d, dout)),
                  full((1, dout))],
        out_specs=out_spec,
        out_shape=out_ty,
    )(out1, out2, wc1, wc2, bc)
    out4 = pl.pallas_call(
        _tc4_body,
        grid=grid,
        in_specs=[row_spec, row_spec, full((d, dout)), full((1, dout))],
        out_specs=out_spec,
        out_shape=out_ty,
    )(out2, gsum, wa, ba)
    return (out3, out4)


# speedup vs baseline: 2.1143x; 1.0136x over previous
import functools

import jax
import jax.numpy as jnp
from jax import lax
from jax.experimental import pallas as pl
from jax.experimental.pallas import tpu as pltpu
from jax.experimental.pallas import tpu_sc as plsc

_NC = 2
_NS = 16
_NW = _NC * _NS
_CHUNK = 64


def _make_sc_gather_sum(n_rows, d, n_pad, rpw0, rpw1):
    rpw_max = max(rpw0, rpw1)
    mesh = plsc.VectorSubcoreMesh(
        core_axis_name="c", subcore_axis_name="s",
        num_cores=_NC, num_subcores=_NS)

    @functools.partial(
        pl.kernel,
        out_type=jax.ShapeDtypeStruct((n_pad, d), jnp.float32),
        mesh=mesh,
        scratch_types=[
            [pltpu.VMEM((rpw_max,), jnp.int32) for _ in range(3)],
            [pltpu.VMEM((_CHUNK, d), jnp.float32) for _ in range(3)],
            [pltpu.VMEM((_CHUNK, d), jnp.float32) for _ in range(3)],
            [pltpu.SemaphoreType.DMA for _ in range(2)],
            [pltpu.SemaphoreType.DMA for _ in range(2)],
        ],
    )
    def sc_gather_sum(out2_hbm, nb0_hbm, nb1_hbm, nb2_hbm, gsum_hbm,
                      idx_all, bufs0, bufs1, sems, st_sems):
        bufs = (bufs0, bufs1)
        cid = lax.axis_index("c")
        sid = lax.axis_index("s")
        rpw = lax.select(cid == 0, rpw0, rpw1)
        base = lax.select(cid == 0, sid * rpw0, _NS * rpw0 + sid * rpw1)
        n_chunks = rpw // _CHUNK
        nbs = (nb0_hbm, nb1_hbm, nb2_hbm)
        for k in range(3):
            pltpu.sync_copy(nbs[k].at[pl.ds(base, rpw_max)], idx_all[k])

        def drain_store(slot):
            pltpu.make_async_copy(
                bufs[slot][0], gsum_hbm.at[pl.ds(0, _CHUNK)],
                st_sems[slot]).wait()

        def fire(ci, slot, first=False):
            for k in (1, 2):
                idx = idx_all[k].at[pl.ds(ci * _CHUNK, _CHUNK)]
                pltpu.async_copy(out2_hbm.at[idx], bufs[slot][k], sems[slot])
            if not first:
                drain_store(slot)
            idx = idx_all[0].at[pl.ds(ci * _CHUNK, _CHUNK)]
            pltpu.async_copy(out2_hbm.at[idx], bufs[slot][0], sems[slot])

        def drain(slot):
            for k in range(3):
                pltpu.make_async_copy(
                    out2_hbm.at[pl.ds(0, _CHUNK)], bufs[slot][k],
                    sems[slot]).wait()

        def process(ci, slot):
            b0, b1, b2 = bufs[slot]

            def row_body(r, rc):
                for c in range(d // 16):
                    sl = pl.ds(c * 16, 16)
                    plsc.addupdate(b0.at[r, sl], b1[r, sl] + b2[r, sl])
                return rc

            lax.fori_loop(0, _CHUNK, row_body, 0)
            pltpu.async_copy(
                b0, gsum_hbm.at[pl.ds(base + ci * _CHUNK, _CHUNK)],
                st_sems[slot])

        fire(0, 0, first=True)
        fire(1, 1, first=True)

        def pair_body(p, carry):
            ci = 2 * p
            drain(0)
            process(ci, 0)

            @pl.when(ci + 2 < n_chunks)
            def _():
                fire(ci + 2, 0)

            drain(1)
            process(ci + 1, 1)

            @pl.when(ci + 3 < n_chunks)
            def _():
                fire(ci + 3, 1)

            return carry

        lax.fori_loop(0, n_chunks // 2, pair_body, 0)
        drain_store(0)
        drain_store(1)

    return sc_gather_sum


def _tc3_body(o1, o2, wc1, wc2, bc, out3):
    out3[...] = (
        jnp.dot(o1[...].astype(jnp.bfloat16), wc1[...],
                preferred_element_type=jnp.float32)
        + jnp.dot(o2[...].astype(jnp.bfloat16), wc2[...],
                  preferred_element_type=jnp.float32)
        + bc[...]
    )


def _tc4_body(o2, g, wa, ba, out4):
    f = ((o2[...] + g[...]) * 0.25).astype(jnp.bfloat16)
    out4[...] = jnp.dot(f, wa[...], preferred_element_type=jnp.float32) + ba[...]


def kernel(out1, out2, neighbour, W_comb, b_comb, W_agg, b_agg):
    n, d = out2.shape
    dout = b_comb.shape[0]

    step = 2 * _CHUNK
    per_pair = ((n + _NS - 1) // _NS + step - 1) // step * step
    rpw1 = max(step, int(round(per_pair * 0.265 / step)) * step)
    rpw0 = per_pair - rpw1
    n_pad = _NS * per_pair
    nbt = jnp.transpose(neighbour.astype(jnp.int32))
    nbt = jnp.pad(nbt, ((0, 0), (0, n_pad + max(rpw0, rpw1) - n)))
    gsum = _make_sc_gather_sum(n, d, n_pad, rpw0, rpw1)(
        out2, nbt[0], nbt[1], nbt[2])

    wcT = jnp.transpose(W_comb[:, :, 0]).astype(jnp.bfloat16)
    wc1 = wcT[:d]
    wc2 = wcT[d:]
    wa = jnp.transpose(W_agg[:, :, 0]).astype(jnp.bfloat16)
    bc = b_comb.reshape(1, dout)
    ba = b_agg.reshape(1, dout)

    blk = 2000
    assert n % blk == 0
    grid = (n // blk,)
    row_spec = pl.BlockSpec((blk, d), lambda i: (i, 0))
    out_spec = pl.BlockSpec((blk, dout), lambda i: (i, 0))
    full = lambda s: pl.BlockSpec(s, lambda i: (0, 0))
    out_ty = jax.ShapeDtypeStruct((n, dout), jnp.float32)
    out3 = pl.pallas_call(
        _tc3_body,
        grid=grid,
        in_specs=[row_spec, row_spec, full((d, dout)), full((d, dout)),
                  full((1, dout))],
        out_specs=out_spec,
        out_shape=out_ty,
    )(out1, out2, wc1, wc2, bc)
    out4 = pl.pallas_call(
        _tc4_body,
        grid=grid,
        in_specs=[row_spec, row_spec, full((d, dout)), full((1, dout))],
        out_specs=out_spec,
        out_shape=out_ty,
    )(out2, gsum, wa, ba)
    return (out3, out4)


# speedup vs baseline: 2.1584x; 1.0209x over previous
import functools

import jax
import jax.numpy as jnp
from jax import lax
from jax.experimental import pallas as pl
from jax.experimental.pallas import tpu as pltpu
from jax.experimental.pallas import tpu_sc as plsc

_NC = 2
_NS = 16
_NW = _NC * _NS
_CHUNK = 32
_SLOTS = 4


def _make_sc_gather_sum(n_rows, d, n_pad, rpw0, rpw1):
    rpw_max = max(rpw0, rpw1)
    mesh = plsc.VectorSubcoreMesh(
        core_axis_name="c", subcore_axis_name="s",
        num_cores=_NC, num_subcores=_NS)

    @functools.partial(
        pl.kernel,
        out_type=jax.ShapeDtypeStruct((n_pad, d), jnp.float32),
        mesh=mesh,
        scratch_types=[
            [pltpu.VMEM((rpw_max,), jnp.int32) for _ in range(3)],
            [[pltpu.VMEM((_CHUNK, d), jnp.float32) for _ in range(3)]
             for _ in range(_SLOTS)],
            [pltpu.SemaphoreType.DMA for _ in range(_SLOTS)],
            [pltpu.SemaphoreType.DMA for _ in range(_SLOTS)],
        ],
    )
    def sc_gather_sum(out2_hbm, nb0_hbm, nb1_hbm, nb2_hbm, gsum_hbm,
                      idx_all, bufs, sems, st_sems):
        cid = lax.axis_index("c")
        sid = lax.axis_index("s")
        rpw = lax.select(cid == 0, rpw0, rpw1)
        base = lax.select(cid == 0, sid * rpw0, _NS * rpw0 + sid * rpw1)
        n_chunks = rpw // _CHUNK
        nbs = (nb0_hbm, nb1_hbm, nb2_hbm)
        for k in range(3):
            pltpu.sync_copy(nbs[k].at[pl.ds(base, rpw_max)], idx_all[k])

        def drain_store(slot):
            pltpu.make_async_copy(
                bufs[slot][0], gsum_hbm.at[pl.ds(0, _CHUNK)],
                st_sems[slot]).wait()

        def fire(ci, slot, first=False):
            for k in (1, 2):
                idx = idx_all[k].at[pl.ds(ci * _CHUNK, _CHUNK)]
                pltpu.async_copy(out2_hbm.at[idx], bufs[slot][k], sems[slot])
            if not first:
                drain_store(slot)
            idx = idx_all[0].at[pl.ds(ci * _CHUNK, _CHUNK)]
            pltpu.async_copy(out2_hbm.at[idx], bufs[slot][0], sems[slot])

        def drain(slot):
            for k in range(3):
                pltpu.make_async_copy(
                    out2_hbm.at[pl.ds(0, _CHUNK)], bufs[slot][k],
                    sems[slot]).wait()

        def process(ci, slot):
            b0, b1, b2 = bufs[slot]

            def row_body(r, rc):
                for c in range(d // 16):
                    sl = pl.ds(c * 16, 16)
                    plsc.addupdate(b0.at[r, sl], b1[r, sl] + b2[r, sl])
                return rc

            lax.fori_loop(0, _CHUNK, row_body, 0)
            pltpu.async_copy(
                b0, gsum_hbm.at[pl.ds(base + ci * _CHUNK, _CHUNK)],
                st_sems[slot])

        for s in range(_SLOTS):
            fire(s, s, first=True)

        def group_body(p, carry):
            for s in range(_SLOTS):
                ci = _SLOTS * p + s
                drain(s)
                process(ci, s)

                @pl.when(ci + _SLOTS < n_chunks)
                def _():
                    fire(ci + _SLOTS, s)

            return carry

        lax.fori_loop(0, n_chunks // _SLOTS, group_body, 0)
        for s in range(_SLOTS):
            drain_store(s)

    return sc_gather_sum


def _tc3_body(o1, o2, wc1, wc2, bc, out3):
    out3[...] = (
        jnp.dot(o1[...].astype(jnp.bfloat16), wc1[...],
                preferred_element_type=jnp.float32)
        + jnp.dot(o2[...].astype(jnp.bfloat16), wc2[...],
                  preferred_element_type=jnp.float32)
        + bc[...]
    )


def _tc4_body(o2, g, wa, ba, out4):
    f = ((o2[...] + g[...]) * 0.25).astype(jnp.bfloat16)
    out4[...] = jnp.dot(f, wa[...], preferred_element_type=jnp.float32) + ba[...]


def kernel(out1, out2, neighbour, W_comb, b_comb, W_agg, b_agg):
    n, d = out2.shape
    dout = b_comb.shape[0]

    step = _SLOTS * _CHUNK
    per_pair = ((n + _NS - 1) // _NS + step - 1) // step * step
    rpw1 = max(step, int(round(per_pair * 0.265 / step)) * step)
    rpw0 = per_pair - rpw1
    n_pad = _NS * per_pair
    nbt = jnp.transpose(neighbour.astype(jnp.int32))
    nbt = jnp.pad(nbt, ((0, 0), (0, n_pad + max(rpw0, rpw1) - n)))
    gsum = _make_sc_gather_sum(n, d, n_pad, rpw0, rpw1)(
        out2, nbt[0], nbt[1], nbt[2])

    wcT = jnp.transpose(W_comb[:, :, 0]).astype(jnp.bfloat16)
    wc1 = wcT[:d]
    wc2 = wcT[d:]
    wa = jnp.transpose(W_agg[:, :, 0]).astype(jnp.bfloat16)
    bc = b_comb.reshape(1, dout)
    ba = b_agg.reshape(1, dout)

    blk = 2000
    assert n % blk == 0
    grid = (n // blk,)
    row_spec = pl.BlockSpec((blk, d), lambda i: (i, 0))
    out_spec = pl.BlockSpec((blk, dout), lambda i: (i, 0))
    full = lambda s: pl.BlockSpec(s, lambda i: (0, 0))
    out_ty = jax.ShapeDtypeStruct((n, dout), jnp.float32)
    out3 = pl.pallas_call(
        _tc3_body,
        grid=grid,
        in_specs=[row_spec, row_spec, full((d, dout)), full((d, dout)),
                  full((1, dout))],
        out_specs=out_spec,
        out_shape=out_ty,
    )(out1, out2, wc1, wc2, bc)
    out4 = pl.pallas_call(
        _tc4_body,
        grid=grid,
        in_specs=[row_spec, row_spec, full((d, dout)), full((1, dout))],
        out_specs=out_spec,
        out_shape=out_ty,
    )(out2, gsum, wa, ba)
    return (out3, out4)
